# baseline (speedup 1.0000x reference)
import functools

import jax
import jax.numpy as jnp
from jax.experimental import pallas as pl
from jax.experimental.pallas import tpu as pltpu

_ROUTING_ITERS = 3


def _gat2_body(h_ref, W0_ref, as0_ref, ad0_ref, W1_ref, as1_ref, ad1_ref,
               mask_ref, out_ref):
    Bb, L, D = h_ref.shape
    mask3 = mask_ref[...][None]
    params = ((W0_ref[...], as0_ref[...], ad0_ref[...]),
              (W1_ref[...], as1_ref[...], ad1_ref[...]))
    h_list = [h_ref[s] for s in range(Bb)]
    for (W, a_s, a_d) in params:
        Wh_list = [jnp.dot(h, W, preferred_element_type=jnp.float32)
                   for h in h_list]
        es3 = jnp.stack(
            [jnp.dot(Wh, a_s, preferred_element_type=jnp.float32)
             for Wh in Wh_list])
        ed3 = jnp.stack(
            [jax.lax.dot_general(a_d, Wh, (((0,), (1,)), ((), ())),
                                 preferred_element_type=jnp.float32)
             for Wh in Wh_list])
        e = es3 + ed3
        e = jnp.where(e >= 0, e, 0.2 * e)
        e = jnp.where(mask3 > 0, e, -1e9)
        m = jnp.max(e, axis=-1, keepdims=True)
        ex = jnp.exp(e - m)
        alpha = ex / jnp.sum(ex, axis=-1, keepdims=True)
        h_list = []
        for s in range(Bb):
            hn = jnp.dot(alpha[s], Wh_list[s],
                         preferred_element_type=jnp.float32)
            h_list.append(
                jnp.where(hn > 0, hn, jnp.exp(jnp.minimum(hn, 0.0)) - 1.0))
    for s in range(Bb):
        out_ref[:, s, :] = h_list[s].astype(out_ref.dtype)


def _caps_body(h_ref, Wc_ref, out_ref, *, P, CIN, COUT, iters):
    Lb = h_ref.shape[0]
    pred_rows = []
    for l in range(Lb):
        h_l = h_ref[l]
        W_l = Wc_ref[l]
        u = jnp.concatenate(
            [jax.lax.dot_general(W_l[p], h_l[:, p * CIN:(p + 1) * CIN],
                                 (((0,), (1,)), ((), ())),
                                 preferred_element_type=jnp.float32)
             for p in range(P)], axis=0)
        Bn = u.shape[1]
        b = jnp.zeros((P, Bn), dtype=jnp.float32)
        v = None
        for _ in range(iters):
            m = jnp.max(b, axis=0, keepdims=True)
            ex = jnp.exp(b - m)
            c = ex / jnp.sum(ex, axis=0, keepdims=True)
            s = jnp.zeros((COUT, Bn), dtype=jnp.float32)
            for p in range(P):
                s = s + c[p:p + 1, :] * u[p * COUT:(p + 1) * COUT, :]
            n2 = jnp.sum(s * s, axis=0, keepdims=True)
            v = (n2 / (1.0 + n2)) * s / jnp.sqrt(n2 + 1e-8)
            rows = [jnp.sum(u[p * COUT:(p + 1) * COUT, :] * v, axis=0,
                            keepdims=True) for p in range(P)]
            b = b + jnp.concatenate(rows, axis=0)
        pred_rows.append(
            jnp.sqrt(jnp.sum(v * v, axis=0, keepdims=True) + 1e-8))
    out_ref[:, 0, :] = jnp.concatenate(pred_rows, axis=0)


def kernel(inputs, W_g0, a_src0, a_dst0, W_g1, a_src1, a_dst1, W_caps,
           adj_mask):
    B, L, D = inputs.shape
    _, P, CIN, COUT = W_caps.shape
    Bb = 16
    maskf = adj_mask.astype(jnp.float32)
    as0 = a_src0.reshape(D, 1)
    ad0 = a_dst0.reshape(D, 1)
    as1 = a_src1.reshape(D, 1)
    ad1 = a_dst1.reshape(D, 1)

    h2 = pl.pallas_call(
        _gat2_body,
        grid=(B // Bb,),
        in_specs=[
            pl.BlockSpec((Bb, L, D), lambda i: (i, 0, 0)),
            pl.BlockSpec((D, D), lambda i: (0, 0)),
            pl.BlockSpec((D, 1), lambda i: (0, 0)),
            pl.BlockSpec((D, 1), lambda i: (0, 0)),
            pl.BlockSpec((D, D), lambda i: (0, 0)),
            pl.BlockSpec((D, 1), lambda i: (0, 0)),
            pl.BlockSpec((D, 1), lambda i: (0, 0)),
            pl.BlockSpec((L, L), lambda i: (0, 0)),
        ],
        out_specs=pl.BlockSpec((L, Bb, D), lambda i: (0, i, 0)),
        out_shape=jax.ShapeDtypeStruct((L, B, D), jnp.bfloat16),
        compiler_params=pltpu.CompilerParams(
            dimension_semantics=("parallel",)),
    )(inputs, W_g0, as0, ad0, W_g1, as1, ad1, maskf)

    Lb = 3
    preds_t = pl.pallas_call(
        functools.partial(_caps_body, P=P, CIN=CIN, COUT=COUT,
                          iters=_ROUTING_ITERS),
        grid=(L // Lb,),
        in_specs=[
            pl.BlockSpec((Lb, B, D), lambda l: (l, 0, 0)),
            pl.BlockSpec((Lb, P, CIN, COUT), lambda l: (l, 0, 0, 0)),
        ],
        out_specs=pl.BlockSpec((Lb, 1, B), lambda l: (l, 0, 0)),
        out_shape=jax.ShapeDtypeStruct((L, 1, B), jnp.float32),
        compiler_params=pltpu.CompilerParams(
            dimension_semantics=("parallel",)),
    )(h2, W_caps)

    return preds_t[:, 0, :].T


# speedup vs baseline: 1.0830x; 1.0830x over previous
import functools

import jax
import jax.numpy as jnp
from jax.experimental import pallas as pl
from jax.experimental.pallas import tpu as pltpu

_ROUTING_ITERS = 3


def _gat2_body(h_ref, W0_ref, as0_ref, ad0_ref, W1_ref, as1_ref, ad1_ref,
               mask_ref, out_ref):
    Bb, L, D = h_ref.shape
    mask3 = mask_ref[...][None]
    params = ((W0_ref[...], as0_ref[...], ad0_ref[...]),
              (W1_ref[...], as1_ref[...], ad1_ref[...]))
    h_list = [h_ref[s] for s in range(Bb)]
    for (W, a_s, a_d) in params:
        Wh_list = [jnp.dot(h, W, preferred_element_type=jnp.float32)
                   for h in h_list]
        es3 = jnp.stack(
            [jnp.dot(Wh, a_s, preferred_element_type=jnp.float32)
             for Wh in Wh_list])
        ed3 = jnp.stack(
            [jax.lax.dot_general(a_d, Wh, (((0,), (1,)), ((), ())),
                                 preferred_element_type=jnp.float32)
             for Wh in Wh_list])
        e = es3 + ed3
        e = jnp.where(e >= 0, e, 0.2 * e)
        e = jnp.where(mask3 > 0, e, -1e9)
        m = jnp.max(e, axis=-1, keepdims=True)
        ex = jnp.exp(e - m)
        alpha = ex / jnp.sum(ex, axis=-1, keepdims=True)
        h_list = []
        for s in range(Bb):
            hn = jnp.dot(alpha[s], Wh_list[s],
                         preferred_element_type=jnp.float32)
            h_list.append(
                jnp.where(hn > 0, hn, jnp.exp(jnp.minimum(hn, 0.0)) - 1.0))
    for s in range(Bb):
        out_ref[:, s, :] = h_list[s]


def _caps_body(h_ref, Wc_ref, out_ref, *, P, CIN, COUT, iters):
    Lb = h_ref.shape[0]
    pred_rows = []
    for l in range(Lb):
        h_l = h_ref[l]
        W_l = Wc_ref[l]
        u = jnp.concatenate(
            [jax.lax.dot_general(W_l[p], h_l[:, p * CIN:(p + 1) * CIN],
                                 (((0,), (1,)), ((), ())),
                                 preferred_element_type=jnp.float32)
             for p in range(P)], axis=0)
        Bn = u.shape[1]
        b = jnp.zeros((P, Bn), dtype=jnp.float32)
        v = None
        for _ in range(iters):
            m = jnp.max(b, axis=0, keepdims=True)
            ex = jnp.exp(b - m)
            c = ex / jnp.sum(ex, axis=0, keepdims=True)
            s = jnp.zeros((COUT, Bn), dtype=jnp.float32)
            for p in range(P):
                s = s + c[p:p + 1, :] * u[p * COUT:(p + 1) * COUT, :]
            n2 = jnp.sum(s * s, axis=0, keepdims=True)
            v = (n2 / (1.0 + n2)) * s / jnp.sqrt(n2 + 1e-8)
            rows = [jnp.sum(u[p * COUT:(p + 1) * COUT, :] * v, axis=0,
                            keepdims=True) for p in range(P)]
            b = b + jnp.concatenate(rows, axis=0)
        pred_rows.append(
            jnp.sqrt(jnp.sum(v * v, axis=0, keepdims=True) + 1e-8))
    out_ref[:, 0, :] = jnp.concatenate(pred_rows, axis=0)


def kernel(inputs, W_g0, a_src0, a_dst0, W_g1, a_src1, a_dst1, W_caps,
           adj_mask):
    B, L, D = inputs.shape
    _, P, CIN, COUT = W_caps.shape
    Bb = 32
    maskf = adj_mask.astype(jnp.float32)
    as0 = a_src0.reshape(D, 1)
    ad0 = a_dst0.reshape(D, 1)
    as1 = a_src1.reshape(D, 1)
    ad1 = a_dst1.reshape(D, 1)

    h2 = pl.pallas_call(
        _gat2_body,
        grid=(B // Bb,),
        in_specs=[
            pl.BlockSpec((Bb, L, D), lambda i: (i, 0, 0)),
            pl.BlockSpec((D, D), lambda i: (0, 0)),
            pl.BlockSpec((D, 1), lambda i: (0, 0)),
            pl.BlockSpec((D, 1), lambda i: (0, 0)),
            pl.BlockSpec((D, D), lambda i: (0, 0)),
            pl.BlockSpec((D, 1), lambda i: (0, 0)),
            pl.BlockSpec((D, 1), lambda i: (0, 0)),
            pl.BlockSpec((L, L), lambda i: (0, 0)),
        ],
        out_specs=pl.BlockSpec((L, Bb, D), lambda i: (0, i, 0)),
        out_shape=jax.ShapeDtypeStruct((L, B, D), jnp.float32),
        compiler_params=pltpu.CompilerParams(
            dimension_semantics=("parallel",)),
    )(inputs, W_g0, as0, ad0, W_g1, as1, ad1, maskf)

    Lb = 8
    preds_t = pl.pallas_call(
        functools.partial(_caps_body, P=P, CIN=CIN, COUT=COUT,
                          iters=_ROUTING_ITERS),
        grid=((L + Lb - 1) // Lb,),
        in_specs=[
            pl.BlockSpec((Lb, B, D), lambda l: (l, 0, 0)),
            pl.BlockSpec((Lb, P, CIN, COUT), lambda l: (l, 0, 0, 0)),
        ],
        out_specs=pl.BlockSpec((Lb, 1, B), lambda l: (l, 0, 0)),
        out_shape=jax.ShapeDtypeStruct((L, 1, B), jnp.float32),
        compiler_params=pltpu.CompilerParams(
            dimension_semantics=("parallel",)),
    )(h2, W_caps)

    return preds_t[:, 0, :].T


# speedup vs baseline: 1.2078x; 1.1152x over previous
import functools

import jax
import jax.numpy as jnp
from jax.experimental import pallas as pl
from jax.experimental.pallas import tpu as pltpu

_ROUTING_ITERS = 3


def _gat2_body(h_ref, W0_ref, ad0_ref, W1_ref, ad1_ref,
               mask_ref, out_ref):
    Bb, L, _ = h_ref.shape
    D = W0_ref.shape[0]
    mask3 = mask_ref[...][None]
    params = ((W0_ref[...], ad0_ref[...]), (W1_ref[...], ad1_ref[...]))
    h_list = [h_ref[s] for s in range(Bb)]
    for (Wx, a_d) in params:
        WhE_list = [jnp.dot(h, Wx, preferred_element_type=jnp.float32)
                    for h in h_list]
        es3 = jnp.stack([WhE[:, D:D + 1] for WhE in WhE_list])
        ed3 = jnp.stack(
            [jax.lax.dot_general(a_d, WhE, (((0,), (1,)), ((), ())),
                                 preferred_element_type=jnp.float32)
             for WhE in WhE_list])
        e = es3 + ed3
        e = jnp.where(e >= 0, e, 0.2 * e)
        e = jnp.where(mask3 > 0, e, -1e9)
        m = jnp.max(e, axis=-1, keepdims=True)
        ex = jnp.exp(e - m)
        r3 = 1.0 / jnp.sum(ex, axis=-1, keepdims=True)
        h_list = []
        for s in range(Bb):
            hn = jnp.dot(ex[s], WhE_list[s],
                         preferred_element_type=jnp.float32)
            hn = hn[:, :D] * r3[s]
            h_list.append(
                jnp.where(hn > 0, hn, jnp.exp(jnp.minimum(hn, 0.0)) - 1.0))
    for s in range(Bb):
        out_ref[:, s, :] = h_list[s]


def _caps_body(h_ref, Wc_ref, out_ref, *, P, CIN, COUT, iters):
    Lb = h_ref.shape[0]
    pred_rows = []
    for l in range(Lb):
        h_l = h_ref[l]
        W_l = Wc_ref[l]
        u = jnp.concatenate(
            [jax.lax.dot_general(W_l[p], h_l[:, p * CIN:(p + 1) * CIN],
                                 (((0,), (1,)), ((), ())),
                                 preferred_element_type=jnp.float32)
             for p in range(P)], axis=0)
        Bn = u.shape[1]
        b = jnp.zeros((P, Bn), dtype=jnp.float32)
        v = None
        for it in range(iters):
            m = jnp.max(b, axis=0, keepdims=True)
            ex = jnp.exp(b - m)
            c = ex / jnp.sum(ex, axis=0, keepdims=True)
            s = jnp.zeros((COUT, Bn), dtype=jnp.float32)
            for p in range(P):
                s = s + c[p:p + 1, :] * u[p * COUT:(p + 1) * COUT, :]
            n2 = jnp.sum(s * s, axis=0, keepdims=True)
            v = (n2 / (1.0 + n2)) * s / jnp.sqrt(n2 + 1e-8)
            if it + 1 < iters:
                rows = [jnp.sum(u[p * COUT:(p + 1) * COUT, :] * v, axis=0,
                                keepdims=True) for p in range(P)]
                b = b + jnp.concatenate(rows, axis=0)
        pred_rows.append(
            jnp.sqrt(jnp.sum(v * v, axis=0, keepdims=True) + 1e-8))
    out_ref[:, 0, :] = jnp.concatenate(pred_rows, axis=0)


def kernel(inputs, W_g0, a_src0, a_dst0, W_g1, a_src1, a_dst1, W_caps,
           adj_mask):
    B, L, D = inputs.shape
    _, P, CIN, COUT = W_caps.shape
    Bb = 32
    maskf = adj_mask.astype(jnp.float32)
    Wx0 = jnp.concatenate([W_g0, (W_g0 @ a_src0)[:, None]], axis=1)
    Wx1 = jnp.concatenate([W_g1, (W_g1 @ a_src1)[:, None]], axis=1)
    ad0 = jnp.pad(a_dst0, (0, 1)).reshape(D + 1, 1)
    ad1 = jnp.pad(a_dst1, (0, 1)).reshape(D + 1, 1)

    h2 = pl.pallas_call(
        _gat2_body,
        grid=(B // Bb,),
        in_specs=[
            pl.BlockSpec((Bb, L, D), lambda i: (i, 0, 0)),
            pl.BlockSpec((D, D + 1), lambda i: (0, 0)),
            pl.BlockSpec((D + 1, 1), lambda i: (0, 0)),
            pl.BlockSpec((D, D + 1), lambda i: (0, 0)),
            pl.BlockSpec((D + 1, 1), lambda i: (0, 0)),
            pl.BlockSpec((L, L), lambda i: (0, 0)),
        ],
        out_specs=pl.BlockSpec((L, Bb, D), lambda i: (0, i, 0)),
        out_shape=jax.ShapeDtypeStruct((L, B, D), jnp.float32),
        compiler_params=pltpu.CompilerParams(
            dimension_semantics=("parallel",)),
    )(inputs, Wx0, ad0, Wx1, ad1, maskf)

    Lb = 8
    preds_t = pl.pallas_call(
        functools.partial(_caps_body, P=P, CIN=CIN, COUT=COUT,
                          iters=_ROUTING_ITERS),
        grid=((L + Lb - 1) // Lb,),
        in_specs=[
            pl.BlockSpec((Lb, B, D), lambda l: (l, 0, 0)),
            pl.BlockSpec((Lb, P, CIN, COUT), lambda l: (l, 0, 0, 0)),
        ],
        out_specs=pl.BlockSpec((Lb, 1, B), lambda l: (l, 0, 0)),
        out_shape=jax.ShapeDtypeStruct((L, 1, B), jnp.float32),
        compiler_params=pltpu.CompilerParams(
            dimension_semantics=("parallel",)),
    )(h2, W_caps)

    return preds_t[:, 0, :].T
